# baseline (speedup 1.0000x reference)
import jax
import jax.numpy as jnp
from jax import lax
from jax.experimental import pallas as pl
from jax.experimental.pallas import tpu as pltpu
from jax.experimental.pallas import tpu_sc as plsc

NPOINT = 2048
N = 65536
B = 8
NS = 16
L = 16
TPG = 4
GROUPS = NS // TPG
SLICE = N // TPG
CHUNKS = SLICE // L
NEG = -3e38


def _shuf(v, perm):
    return v.at[perm].get(mode="promise_in_bounds")


def _bcast_max(v, lane):
    for k in (1, 2, 4, 8):
        v = jnp.maximum(v, _shuf(v, lane ^ k))
    return v


def _bcast_min(v, lane):
    for k in (1, 2, 4, 8):
        v = jnp.minimum(v, _shuf(v, lane ^ k))
    return v


def _pick(mask, v, lane):
    return _bcast_max(jnp.where(mask, v, NEG), lane)


def _fps_sc_body(x_hbm, y_hbm, z_hbm, out_hbm, mail_hbm, maili_hbm,
                 xt, yt, zt, dt, outbuf, stagef, stagei, tmp, grpf, grpi):
    c = lax.axis_index("c")
    s = lax.axis_index("s")
    g = s // TPG
    r = s % TPG
    batch = c * GROUPS + g
    base = r * SLICE
    lane = lax.iota(jnp.int32, L)

    pltpu.sync_copy(x_hbm.at[batch, pl.ds(base, SLICE)], xt)
    pltpu.sync_copy(y_hbm.at[batch, pl.ds(base, SLICE)], yt)
    pltpu.sync_copy(z_hbm.at[batch, pl.ds(base, SLICE)], zt)

    def initc(ci, carry):
        dt[pl.ds(ci * L, L)] = jnp.full((L,), 1e10, jnp.float32)
        return carry
    lax.fori_loop(0, CHUNKS, initc, 0)

    pltpu.sync_copy(x_hbm.at[batch, pl.ds(0, L)], tmp)
    cx0 = _pick(lane == 0, tmp[...], lane)
    pltpu.sync_copy(y_hbm.at[batch, pl.ds(0, L)], tmp)
    cy0 = _pick(lane == 0, tmp[...], lane)
    pltpu.sync_copy(z_hbm.at[batch, pl.ds(0, L)], tmp)
    cz0 = _pick(lane == 0, tmp[...], lane)

    def it(i, carry):
        f, cx, cy, cz, fbuf = carry
        fbuf = jnp.where(lane == (i % L), f, fbuf)

        @pl.when(i % L == L - 1)
        def _():
            outbuf[pl.ds(i - (L - 1), L)] = fbuf

        def chunk(ci, pcarry):
            mv, iv, xv, yv, zv = pcarry
            sl = pl.ds(ci * L, L)
            xc = xt[sl]
            yc = yt[sl]
            zc = zt[sl]
            dc = dt[sl]
            dx = xc - cx
            dy = yc - cy
            dz = zc - cz
            d = (dx * dx + dz * dz) + dy * dy
            nd = jnp.minimum(dc, d)
            dt[sl] = nd
            m = nd > mv
            gi = (base + ci * L) + lane
            return (jnp.where(m, nd, mv), jnp.where(m, gi, iv),
                    jnp.where(m, xc, xv), jnp.where(m, yc, yv),
                    jnp.where(m, zc, zv))

        zf = jnp.zeros((L,), jnp.float32)
        mv, iv, xv, yv, zv = lax.fori_loop(
            0, CHUNKS, chunk,
            (jnp.full((L,), -1.0, jnp.float32), jnp.zeros((L,), jnp.int32),
             zf, zf, zf))

        stagef[0] = mv
        stagef[1] = xv
        stagef[2] = yv
        stagef[3] = zv
        stagei[...] = iv
        pltpu.sync_copy(stagef, mail_hbm.at[c, s])
        pltpu.sync_copy(stagei, maili_hbm.at[c, s])
        plsc.subcore_barrier()
        pltpu.sync_copy(mail_hbm.at[c, pl.ds(g * TPG, TPG)], grpf)
        pltpu.sync_copy(maili_hbm.at[c, pl.ds(g * TPG, TPG)], grpi)
        plsc.subcore_barrier()

        bm = grpf[0, 0]
        bi = grpi[0]
        bx = grpf[0, 1]
        by = grpf[0, 2]
        bz = grpf[0, 3]
        for t in range(1, TPG):
            m_ = grpf[t, 0]
            i_ = grpi[t]
            better = (m_ > bm) | ((m_ == bm) & (i_ < bi))
            bx = jnp.where(better, grpf[t, 1], bx)
            by = jnp.where(better, grpf[t, 2], by)
            bz = jnp.where(better, grpf[t, 3], bz)
            bm = jnp.where(better, m_, bm)
            bi = jnp.where(better, i_, bi)
        gmax = _bcast_max(bm, lane)
        nf = _bcast_min(jnp.where(bm == gmax, bi, N), lane)
        win = bi == nf
        ncx = _pick(win, bx, lane)
        ncy = _pick(win, by, lane)
        ncz = _pick(win, bz, lane)
        return nf, ncx, ncy, ncz, fbuf

    f0 = jnp.zeros((L,), jnp.int32)
    lax.fori_loop(0, NPOINT, it, (f0, cx0, cy0, cz0, f0))

    @pl.when(r == 0)
    def _():
        pltpu.sync_copy(outbuf, out_hbm.at[batch])


def kernel(pts):
    ptsT = jnp.transpose(pts, (2, 0, 1))
    x, y, z = ptsT[0], ptsT[1], ptsT[2]
    mesh = plsc.VectorSubcoreMesh(core_axis_name="c", subcore_axis_name="s")
    fn = pl.kernel(
        _fps_sc_body,
        out_type=(jax.ShapeDtypeStruct((B, NPOINT), jnp.int32),
                  jax.ShapeDtypeStruct((2, NS, 4, L), jnp.float32),
                  jax.ShapeDtypeStruct((2, NS, L), jnp.int32)),
        mesh=mesh,
        scratch_types=[
            pltpu.VMEM((SLICE,), jnp.float32),
            pltpu.VMEM((SLICE,), jnp.float32),
            pltpu.VMEM((SLICE,), jnp.float32),
            pltpu.VMEM((SLICE,), jnp.float32),
            pltpu.VMEM((NPOINT,), jnp.int32),
            pltpu.VMEM((4, L), jnp.float32),
            pltpu.VMEM((L,), jnp.int32),
            pltpu.VMEM((L,), jnp.float32),
            pltpu.VMEM((TPG, 4, L), jnp.float32),
            pltpu.VMEM((TPG, L), jnp.int32),
        ],
    )
    out, _mf, _mi = fn(x, y, z)
    return out


# speedup vs baseline: 1.3544x; 1.3544x over previous
import jax
import jax.numpy as jnp
from jax import lax
from jax.experimental import pallas as pl
from jax.experimental.pallas import tpu as pltpu
from jax.experimental.pallas import tpu_sc as plsc

NPOINT = 2048
N = 65536
B = 8
CHUNK = 512

B_TC = 6


def _fps_tc_body(x_ref, y_ref, z_ref, out_ref, dists_ref):
    nchunks = N // CHUNK
    dists_ref[...] = jnp.full((B_TC, N), 1e10, dtype=jnp.float32)
    outpos = lax.broadcasted_iota(jnp.int32, (B_TC, NPOINT), 1)
    chunk_lane = lax.broadcasted_iota(jnp.int32, (B_TC, CHUNK), 1)
    out_ref[...] = jnp.zeros((B_TC, NPOINT), jnp.int32)

    def it(i, carry):
        f, cx, cy, cz = carry
        out_ref[...] = jnp.where(outpos == i, f, out_ref[...])

        mv = jnp.full((B_TC, CHUNK), -1.0, jnp.float32)
        iv = jnp.zeros((B_TC, CHUNK), jnp.int32)
        xv = jnp.zeros((B_TC, CHUNK), jnp.float32)
        yv = jnp.zeros((B_TC, CHUNK), jnp.float32)
        zv = jnp.zeros((B_TC, CHUNK), jnp.float32)

        for c in range(nchunks):
            sl = pl.ds(c * CHUNK, CHUNK)
            xc = x_ref[:, sl]
            yc = y_ref[:, sl]
            zc = z_ref[:, sl]
            dx = xc - cx
            dy = yc - cy
            dz = zc - cz
            d = (dx * dx + dz * dz) + dy * dy
            nd = jnp.minimum(dists_ref[:, sl], d)
            dists_ref[:, sl] = nd
            m = nd > mv
            mv = jnp.where(m, nd, mv)
            iv = jnp.where(m, chunk_lane + (c * CHUNK), iv)
            xv = jnp.where(m, xc, xv)
            yv = jnp.where(m, yc, yv)
            zv = jnp.where(m, zc, zv)

        gmax = jnp.max(mv, axis=1, keepdims=True)
        nf = jnp.min(jnp.where(mv == gmax, iv, N), axis=1, keepdims=True)
        win = iv == nf
        ncx = jnp.sum(jnp.where(win, xv, 0.0), axis=1, keepdims=True)
        ncy = jnp.sum(jnp.where(win, yv, 0.0), axis=1, keepdims=True)
        ncz = jnp.sum(jnp.where(win, zv, 0.0), axis=1, keepdims=True)
        return nf, ncx, ncy, ncz

    f0 = jnp.zeros((B_TC, 1), jnp.int32)
    lax.fori_loop(
        0, NPOINT, it,
        (f0, x_ref[:, 0:1], y_ref[:, 0:1], z_ref[:, 0:1]),
    )


NS = 16
L = 16
SLICE_SC = N // NS
CHUNKS_SC = SLICE_SC // L
NEG = -3e38


def _shuf(v, perm):
    return v.at[perm].get(mode="promise_in_bounds")


def _bcast_max(v, lane):
    for k in (1, 2, 4, 8):
        v = jnp.maximum(v, _shuf(v, lane ^ k))
    return v


def _bcast_min(v, lane):
    for k in (1, 2, 4, 8):
        v = jnp.minimum(v, _shuf(v, lane ^ k))
    return v


def _pick(mask, v, lane):
    return _bcast_max(jnp.where(mask, v, NEG), lane)


def _fps_sc_body(x_hbm, y_hbm, z_hbm, out_hbm, mail_hbm, maili_hbm,
                 xt, yt, zt, dt, outbuf, stagef, stagei, tmp, grpf, grpi):
    c = lax.axis_index("c")
    s = lax.axis_index("s")
    batch = c
    base = s * SLICE_SC
    lane = lax.iota(jnp.int32, L)

    pltpu.sync_copy(x_hbm.at[batch, pl.ds(base, SLICE_SC)], xt)
    pltpu.sync_copy(y_hbm.at[batch, pl.ds(base, SLICE_SC)], yt)
    pltpu.sync_copy(z_hbm.at[batch, pl.ds(base, SLICE_SC)], zt)

    def initc(ci, carry):
        dt[pl.ds(ci * L, L)] = jnp.full((L,), 1e10, jnp.float32)
        return carry
    lax.fori_loop(0, CHUNKS_SC, initc, 0)

    pltpu.sync_copy(x_hbm.at[batch, pl.ds(0, L)], tmp)
    cx0 = _pick(lane == 0, tmp[...], lane)
    pltpu.sync_copy(y_hbm.at[batch, pl.ds(0, L)], tmp)
    cy0 = _pick(lane == 0, tmp[...], lane)
    pltpu.sync_copy(z_hbm.at[batch, pl.ds(0, L)], tmp)
    cz0 = _pick(lane == 0, tmp[...], lane)

    def it(i, carry):
        f, cx, cy, cz, fbuf = carry
        fbuf = jnp.where(lane == (i % L), f, fbuf)

        @pl.when(i % L == L - 1)
        def _():
            outbuf[pl.ds(i - (L - 1), L)] = fbuf

        def chunk(ci, pcarry):
            mv, iv, xv, yv, zv = pcarry
            sl = pl.ds(ci * L, L)
            xc = xt[sl]
            yc = yt[sl]
            zc = zt[sl]
            dc = dt[sl]
            dx = xc - cx
            dy = yc - cy
            dz = zc - cz
            d = (dx * dx + dz * dz) + dy * dy
            nd = jnp.minimum(dc, d)
            dt[sl] = nd
            m = nd > mv
            gi = (base + ci * L) + lane
            return (jnp.where(m, nd, mv), jnp.where(m, gi, iv),
                    jnp.where(m, xc, xv), jnp.where(m, yc, yv),
                    jnp.where(m, zc, zv))

        zf = jnp.zeros((L,), jnp.float32)
        mv, iv, xv, yv, zv = lax.fori_loop(
            0, CHUNKS_SC, chunk,
            (jnp.full((L,), -1.0, jnp.float32), jnp.zeros((L,), jnp.int32),
             zf, zf, zf))

        stagef[0] = mv
        stagef[1] = xv
        stagef[2] = yv
        stagef[3] = zv
        stagei[...] = iv
        pltpu.sync_copy(stagef, mail_hbm.at[c, s])
        pltpu.sync_copy(stagei, maili_hbm.at[c, s])
        plsc.subcore_barrier()
        pltpu.sync_copy(mail_hbm.at[c], grpf)
        pltpu.sync_copy(maili_hbm.at[c], grpi)
        plsc.subcore_barrier()

        bm = grpf[0, 0]
        bi = grpi[0]
        bx = grpf[0, 1]
        by = grpf[0, 2]
        bz = grpf[0, 3]
        for t in range(1, NS):
            m_ = grpf[t, 0]
            i_ = grpi[t]
            better = (m_ > bm) | ((m_ == bm) & (i_ < bi))
            bx = jnp.where(better, grpf[t, 1], bx)
            by = jnp.where(better, grpf[t, 2], by)
            bz = jnp.where(better, grpf[t, 3], bz)
            bm = jnp.where(better, m_, bm)
            bi = jnp.where(better, i_, bi)
        gmax = _bcast_max(bm, lane)
        nf = _bcast_min(jnp.where(bm == gmax, bi, N), lane)
        win = bi == nf
        ncx = _pick(win, bx, lane)
        ncy = _pick(win, by, lane)
        ncz = _pick(win, bz, lane)
        return nf, ncx, ncy, ncz, fbuf

    f0 = jnp.zeros((L,), jnp.int32)
    lax.fori_loop(0, NPOINT, it, (f0, cx0, cy0, cz0, f0))

    @pl.when(s == 0)
    def _():
        pltpu.sync_copy(outbuf, out_hbm.at[batch])


def kernel(pts):
    ptsT = jnp.transpose(pts, (2, 0, 1))
    x, y, z = ptsT[0], ptsT[1], ptsT[2]

    mesh = plsc.VectorSubcoreMesh(core_axis_name="c", subcore_axis_name="s")
    sc_fn = pl.kernel(
        _fps_sc_body,
        out_type=(jax.ShapeDtypeStruct((2, NPOINT), jnp.int32),
                  jax.ShapeDtypeStruct((2, NS, 4, L), jnp.float32),
                  jax.ShapeDtypeStruct((2, NS, L), jnp.int32)),
        mesh=mesh,
        scratch_types=[
            pltpu.VMEM((SLICE_SC,), jnp.float32),
            pltpu.VMEM((SLICE_SC,), jnp.float32),
            pltpu.VMEM((SLICE_SC,), jnp.float32),
            pltpu.VMEM((SLICE_SC,), jnp.float32),
            pltpu.VMEM((NPOINT,), jnp.int32),
            pltpu.VMEM((4, L), jnp.float32),
            pltpu.VMEM((L,), jnp.int32),
            pltpu.VMEM((L,), jnp.float32),
            pltpu.VMEM((NS, 4, L), jnp.float32),
            pltpu.VMEM((NS, L), jnp.int32),
        ],
    )
    out_sc, _mf, _mi = sc_fn(x[B_TC:], y[B_TC:], z[B_TC:])

    spec = pl.BlockSpec((B_TC, N), lambda: (0, 0))
    out_tc = pl.pallas_call(
        _fps_tc_body,
        in_specs=[spec, spec, spec],
        out_specs=pl.BlockSpec((B_TC, NPOINT), lambda: (0, 0)),
        out_shape=jax.ShapeDtypeStruct((B_TC, NPOINT), jnp.int32),
        scratch_shapes=[pltpu.VMEM((B_TC, N), jnp.float32)],
    )(x[:B_TC], y[:B_TC], z[:B_TC])

    return jnp.concatenate([out_tc, out_sc], axis=0)
